# Initial kernel scaffold; baseline (speedup 1.0000x reference)
import functools

import jax
import jax.numpy as jnp
from jax import lax
from jax.experimental import pallas as pl
from jax.experimental.pallas import tpu as pltpu
from jax.experimental.pallas import tpu_sc as plsc


@functools.lru_cache(maxsize=None)
def _build(B, T, V, D):
    info = plsc.get_sparse_core_info()
    NC, NS = info.num_cores, info.num_subcores
    NW = NC * NS
    assert B % NW == 0
    bpw = B // NW
    CB = 4
    assert bpw % CB == 0
    nchunk = bpw // CB
    ROWS = CB * T
    mesh = plsc.VectorSubcoreMesh(core_axis_name="c", subcore_axis_name="s")

    @functools.partial(
        pl.kernel,
        mesh=mesh,
        out_type=jax.ShapeDtypeStruct((B * T, D), jnp.float32),
        scratch_types=[
            pltpu.VMEM((T, D), jnp.float32),
            pltpu.VMEM((ROWS,), jnp.int32),
            pltpu.VMEM((ROWS, D), jnp.float32),
            pltpu.SemaphoreType.DMA,
        ],
    )
    def k(x_hbm, tok_hbm, pos_hbm, out_hbm, pos_v, idx_v, rows_v, sem):
        wid = lax.axis_index("s") * NC + lax.axis_index("c")
        base_row = wid * (bpw * T)
        pltpu.sync_copy(pos_hbm, pos_v)

        def chunk_body(g, carry):
            row0 = base_row + g * ROWS
            pltpu.sync_copy(x_hbm.at[pl.ds(row0, ROWS)], idx_v)
            pltpu.async_copy(tok_hbm.at[idx_v], rows_v, sem).wait()

            def add_body(t, c2):
                for c in range(0, D, 16):
                    vp = pos_v[t, pl.ds(c, 16)]
                    for j in range(CB):
                        r = j * T + t
                        rows_v[r, pl.ds(c, 16)] = rows_v[r, pl.ds(c, 16)] + vp
                return c2

            lax.fori_loop(0, T, add_body, 0)
            pltpu.sync_copy(rows_v, out_hbm.at[pl.ds(row0, ROWS)])
            return carry

        lax.fori_loop(0, nchunk, chunk_body, 0)

    return k


def kernel(x, token_table, pos_table):
    B, T = x.shape
    V, D = token_table.shape
    xf = x.reshape(-1).astype(jnp.int32)
    out = _build(B, T, V, D)(xf, token_table, pos_table)
    return out.reshape(B, T, D)


# speedup vs baseline: 3.7005x; 3.7005x over previous
import functools

import jax
import jax.numpy as jnp
from jax import lax
from jax.experimental import pallas as pl
from jax.experimental.pallas import tpu as pltpu
from jax.experimental.pallas import tpu_sc as plsc


@functools.lru_cache(maxsize=None)
def _build(B, T, V, D):
    info = plsc.get_sparse_core_info()
    NC, NS = info.num_cores, info.num_subcores
    NW = NC * NS
    assert B % NW == 0
    bpw = B // NW
    CB = 4
    assert bpw % CB == 0
    nchunk = bpw // CB
    ROWS = CB * T
    mesh = plsc.VectorSubcoreMesh(core_axis_name="c", subcore_axis_name="s")

    @functools.partial(
        pl.kernel,
        mesh=mesh,
        out_type=jax.ShapeDtypeStruct((B * T, D), jnp.float32),
        scratch_types=[
            pltpu.VMEM((T, D), jnp.float32),
            pltpu.VMEM((ROWS,), jnp.int32),
            pltpu.VMEM((ROWS, D), jnp.float32),
            pltpu.SemaphoreType.DMA,
        ],
        compiler_params=pltpu.CompilerParams(use_tc_tiling_on_sc=False),
    )
    def k(x_hbm, tok_hbm, pos_hbm, out_hbm, pos_v, idx_v, rows_v, sem):
        wid = lax.axis_index("s") * NC + lax.axis_index("c")
        base_row = wid * (bpw * T)
        pltpu.sync_copy(pos_hbm, pos_v)

        def chunk_body(g, carry):
            row0 = base_row + g * ROWS
            pltpu.sync_copy(x_hbm.at[pl.ds(row0, ROWS)], idx_v)
            pltpu.async_copy(tok_hbm.at[idx_v], rows_v, sem).wait()

            def add_body(t, c2):
                for c in range(0, D, 16):
                    vp = pos_v[t, pl.ds(c, 16)]
                    for j in range(CB):
                        r = j * T + t
                        rows_v[r, pl.ds(c, 16)] = rows_v[r, pl.ds(c, 16)] + vp
                return c2

            lax.fori_loop(0, T, add_body, 0)
            pltpu.sync_copy(rows_v, out_hbm.at[pl.ds(row0, ROWS)])
            return carry

        lax.fori_loop(0, nchunk, chunk_body, 0)

    return k


def kernel(x, token_table, pos_table):
    B, T = x.shape
    V, D = token_table.shape
    xf = x.reshape(-1).astype(jnp.int32)
    out = _build(B, T, V, D)(xf, token_table, pos_table)
    return out.reshape(B, T, D)
